# Initial kernel scaffold; baseline (speedup 1.0000x reference)
import jax
import jax.numpy as jnp
from jax.experimental import pallas as pl


def kernel(query_features, llm_features, edge_index, edge_attr, edge_mask, visible_mask, Wq, bq, Wl, bl):
    raise NotImplementedError("write your pallas kernel here")


# speedup vs baseline: 21.6818x; 21.6818x over previous
import functools

import jax
import jax.numpy as jnp
from jax import lax
from jax.experimental import pallas as pl
from jax.experimental.pallas import tpu as pltpu
from jax.experimental.pallas import tpu_sc as plsc

NQ = 50000
NL = 64
H = 64
E = 800000


Q_BLOCK = 2000


def _score_body(qf, lf, wq, bq, wl, bl, out):
    qh = jnp.dot(qf[...], wq[...], preferred_element_type=jnp.float32) + bq[...]
    qn = qh / jnp.maximum(jnp.sqrt(jnp.sum(qh * qh, axis=1, keepdims=True)), 1e-12)
    lh = jnp.dot(lf[...], wl[...], preferred_element_type=jnp.float32) + bl[...]
    ln = lh / jnp.maximum(jnp.sqrt(jnp.sum(lh * lh, axis=1, keepdims=True)), 1e-12)
    s = jnp.dot(qn, ln.T, preferred_element_type=jnp.float32)
    out[...] = jax.nn.sigmoid(s)


def _score_matrix(qf, lf, wq, bq, wl, bl):
    qd = qf.shape[1]
    ld = lf.shape[1]
    return pl.pallas_call(
        _score_body,
        grid=(NQ // Q_BLOCK,),
        in_specs=[
            pl.BlockSpec((Q_BLOCK, qd), lambda i: (i, 0)),
            pl.BlockSpec((NL, ld), lambda i: (0, 0)),
            pl.BlockSpec((qd, H), lambda i: (0, 0)),
            pl.BlockSpec((1, H), lambda i: (0, 0)),
            pl.BlockSpec((ld, H), lambda i: (0, 0)),
            pl.BlockSpec((1, H), lambda i: (0, 0)),
        ],
        out_specs=pl.BlockSpec((Q_BLOCK, H), lambda i: (i, 0)),
        out_shape=jax.ShapeDtypeStruct((NQ, H), jnp.float32),
    )(qf, lf, wq, bq.reshape(1, H), wl, bl.reshape(1, H))


NC, NS = 2, 16
NW = NC * NS
ROW = 128
GSIZE = 14
GROUPS = 14
NROW_W = GROUPS * GSIZE
CHUNK = NROW_W * ROW
E_PAD = NW * CHUNK

_mesh = plsc.VectorSubcoreMesh(
    core_axis_name="c", subcore_axis_name="s", num_cores=NC, num_subcores=NS
)


@functools.partial(
    pl.kernel,
    out_type=jax.ShapeDtypeStruct((NW, NROW_W, ROW), jnp.float32),
    mesh=_mesh,
    scratch_types=[
        pltpu.VMEM((CHUNK,), jnp.int32),
        pltpu.VMEM((CHUNK,), jnp.int32),
        pltpu.VMEM((CHUNK,), jnp.int32),
        pltpu.VMEM((NROW_W, ROW), jnp.float32),
        pltpu.SemaphoreType.DMA,
    ],
)
def _gather_scores(p_hbm, src_hbm, dst_hbm, out_hbm, src_v, dst_v, idx_v, val_v, sem):
    wid = lax.axis_index("s") * NC + lax.axis_index("c")
    base = wid * CHUNK
    pltpu.sync_copy(src_hbm.at[pl.ds(base, CHUNK)], src_v)
    pltpu.sync_copy(dst_hbm.at[pl.ds(base, CHUNK)], dst_v)

    def idx_body(i, _):
        off = i * 16
        s = src_v[pl.ds(off, 16)]
        d = dst_v[pl.ds(off, 16)]
        idx_v[pl.ds(off, 16)] = s * H + d - NQ
        return 0

    lax.fori_loop(0, CHUNK // 16, idx_body, 0)

    def g_body(g, _):
        descs = []
        for b in range(GSIZE):
            r = g * GSIZE + b
            descs.append(
                pltpu.async_copy(
                    p_hbm.at[idx_v.at[pl.ds(r * ROW, ROW)]],
                    val_v.at[r],
                    sem,
                )
            )
        for dsc in descs:
            dsc.wait()
        return 0

    lax.fori_loop(0, GROUPS, g_body, 0)

    pltpu.sync_copy(val_v, out_hbm.at[wid])


def kernel(query_features, llm_features, edge_index, edge_attr, edge_mask, visible_mask, Wq, bq, Wl, bl):
    p = _score_matrix(query_features, llm_features, Wq, bq, Wl, bl)
    p_flat = p.reshape(-1)
    src = edge_index[0].astype(jnp.int32)
    dst = edge_index[1].astype(jnp.int32)
    pad = E_PAD - E
    src_p = jnp.pad(src, (0, pad))
    dst_p = jnp.pad(dst, (0, pad), constant_values=NQ)
    out2d = _gather_scores(p_flat, src_p, dst_p)
    return out2d.reshape(-1)[:E]


# speedup vs baseline: 30.3068x; 1.3978x over previous
import functools

import jax
import jax.numpy as jnp
from jax import lax
from jax.experimental import pallas as pl
from jax.experimental.pallas import tpu as pltpu
from jax.experimental.pallas import tpu_sc as plsc

NQ = 50000
NL = 64
H = 64
E = 800000


Q_BLOCK = 2000


def _score_body(qf, lf, wq, bq, wl, bl, out):
    qh = jnp.dot(qf[...], wq[...], preferred_element_type=jnp.float32) + bq[...]
    qn = qh / jnp.maximum(jnp.sqrt(jnp.sum(qh * qh, axis=1, keepdims=True)), 1e-12)
    lh = jnp.dot(lf[...], wl[...], preferred_element_type=jnp.float32) + bl[...]
    ln = lh / jnp.maximum(jnp.sqrt(jnp.sum(lh * lh, axis=1, keepdims=True)), 1e-12)
    w2 = jnp.concatenate([ln.T, jnp.zeros((H, 128 - NL), jnp.float32)], axis=1)
    s = jnp.dot(qn, w2, preferred_element_type=jnp.float32)
    out[...] = jax.nn.sigmoid(s)


def _score_matrix(qf, lf, wq, bq, wl, bl):
    qd = qf.shape[1]
    ld = lf.shape[1]
    return pl.pallas_call(
        _score_body,
        grid=(NQ // Q_BLOCK,),
        in_specs=[
            pl.BlockSpec((Q_BLOCK, qd), lambda i: (i, 0)),
            pl.BlockSpec((NL, ld), lambda i: (0, 0)),
            pl.BlockSpec((qd, H), lambda i: (0, 0)),
            pl.BlockSpec((1, H), lambda i: (0, 0)),
            pl.BlockSpec((ld, H), lambda i: (0, 0)),
            pl.BlockSpec((1, H), lambda i: (0, 0)),
        ],
        out_specs=pl.BlockSpec((Q_BLOCK, 128), lambda i: (i, 0)),
        out_shape=jax.ShapeDtypeStruct((NQ, 128), jnp.float32),
    )(qf, lf, wq, bq.reshape(1, H), wl, bl.reshape(1, H))


NC, NS = 2, 16
NW = NC * NS
ROW = 128
GSIZE = 14
GROUPS = 14
NROW_W = GROUPS * GSIZE
CHUNK = NROW_W * ROW
E_PAD = NW * CHUNK


E_ROWS = E // 128
PAD_ROWS = E_PAD // 128
IDX_GRID = 16
IDX_BLOCK = PAD_ROWS // IDX_GRID


def _idx_body(src, dst, out):
    i = pl.program_id(0)
    rows = lax.broadcasted_iota(jnp.int32, (IDX_BLOCK, 128), 0) + i * IDX_BLOCK
    flat = src[0] * 128 + dst[0] - NQ
    out[...] = jnp.where(rows < E_ROWS, flat, 0)


def _flat_indices(edge_index):
    ei = edge_index.astype(jnp.int32).reshape(2, E_ROWS, 128)
    return pl.pallas_call(
        _idx_body,
        grid=(IDX_GRID,),
        in_specs=[
            pl.BlockSpec((1, IDX_BLOCK, 128), lambda i: (0, i, 0)),
            pl.BlockSpec((1, IDX_BLOCK, 128), lambda i: (1, i, 0)),
        ],
        out_specs=pl.BlockSpec((IDX_BLOCK, 128), lambda i: (i, 0)),
        out_shape=jax.ShapeDtypeStruct((PAD_ROWS, 128), jnp.int32),
    )(ei, ei)


_mesh = plsc.VectorSubcoreMesh(
    core_axis_name="c", subcore_axis_name="s", num_cores=NC, num_subcores=NS
)


@functools.partial(
    pl.kernel,
    out_type=jax.ShapeDtypeStruct((NW, NROW_W, ROW), jnp.float32),
    mesh=_mesh,
    scratch_types=[
        pltpu.VMEM((CHUNK,), jnp.int32),
        pltpu.VMEM((NROW_W, ROW), jnp.float32),
        pltpu.SemaphoreType.DMA,
    ],
)
def _gather_scores(p_hbm, idx_hbm, out_hbm, idx_v, val_v, sem):
    wid = lax.axis_index("s") * NC + lax.axis_index("c")
    base = wid * CHUNK
    pltpu.sync_copy(idx_hbm.at[pl.ds(base, CHUNK)], idx_v)

    def g_body(g, _):
        for b in range(GSIZE):
            r = g * GSIZE + b
            pltpu.async_copy(
                p_hbm.at[idx_v.at[pl.ds(r * ROW, ROW)]],
                val_v.at[r],
                sem,
            )
        return 0

    lax.fori_loop(0, GROUPS, g_body, 0)
    pltpu.make_async_copy(out_hbm.at[wid], val_v, sem).wait()

    pltpu.sync_copy(val_v, out_hbm.at[wid])


def kernel(query_features, llm_features, edge_index, edge_attr, edge_mask, visible_mask, Wq, bq, Wl, bl):
    p = _score_matrix(query_features, llm_features, Wq, bq, Wl, bl)
    idx = _flat_indices(edge_index)
    out3d = _gather_scores(p.reshape(-1), idx.reshape(-1))
    return out3d.reshape(-1)[:E]
